# baseline (speedup 1.0000x reference)
import jax
import jax.numpy as jnp
from jax.experimental import pallas as pl

D_FEAT = 64
EMBED_DIM = 32
D_OUT = D_FEAT + EMBED_DIM
NUM_GROUPS = 19
MAX_Z = 120

BLOCK = 36864
CHUNK = 6144


def _body(z_ref, gm_ref, featT_ref, tableT_ref, outT_ref):
    outT_ref[:D_FEAT, :] = featT_ref[...]

    z = z_ref[...][None, :]
    gm = gm_ref[...]

    grows = jax.lax.broadcasted_iota(jnp.int32, (NUM_GROUPS, MAX_Z), 0)
    ohgT = (gm == grows).astype(jnp.float32)
    fusedT = jnp.dot(tableT_ref[...], ohgT,
                     preferred_element_type=jnp.float32)

    zrows = jax.lax.broadcasted_iota(jnp.int32, (MAX_Z, CHUNK), 0)
    for c in range(BLOCK // CHUNK):
        zc = z[:, c * CHUNK:(c + 1) * CHUNK]
        ohzT = (zc == zrows).astype(jnp.float32)
        outT_ref[D_FEAT:, c * CHUNK:(c + 1) * CHUNK] = jnp.dot(
            fusedT, ohzT, preferred_element_type=jnp.float32)


def kernel(per_atom_property_tensor, atomic_numbers, table, group_map):
    n = per_atom_property_tensor.shape[0]
    grid = pl.cdiv(n, BLOCK)
    featT = per_atom_property_tensor.T
    tableT = table.T
    z1 = atomic_numbers.astype(jnp.int32)
    gm2 = group_map.astype(jnp.int32).reshape(1, MAX_Z)
    outT = pl.pallas_call(
        _body,
        grid=(grid,),
        in_specs=[
            pl.BlockSpec((BLOCK,), lambda i: (i,)),
            pl.BlockSpec((1, MAX_Z), lambda i: (0, 0)),
            pl.BlockSpec((D_FEAT, BLOCK), lambda i: (0, i)),
            pl.BlockSpec((EMBED_DIM, NUM_GROUPS), lambda i: (0, 0)),
        ],
        out_specs=pl.BlockSpec((D_OUT, BLOCK), lambda i: (0, i)),
        out_shape=jax.ShapeDtypeStruct((D_OUT, n), jnp.float32),
    )(z1, gm2, featT, tableT)
    return outT.T
